# Initial kernel scaffold; baseline (speedup 1.0000x reference)
import jax
import jax.numpy as jnp
from jax.experimental import pallas as pl


def kernel(h, r, t, batch_type, ent_table, rel_table):
    raise NotImplementedError("write your pallas kernel here")


# speedup vs baseline: 2.2222x; 2.2222x over previous
import functools

import jax
import jax.numpy as jnp
from jax import lax
from jax.experimental import pallas as pl
from jax.experimental.pallas import tpu as pltpu
from jax.experimental.pallas import tpu_sc as plsc

NUM_ENT = 1000000
NUM_REL = 1000
ENT_DIM = 128
REL_DIM = 384
GATHER_REL = 384
EMBEDDING_RANGE = 0.109375
GAMMA = 12.0
PI = 3.141592653589793

NC = 2
NS = 16
NW = NC * NS
CHUNK = 128


def _sc_gather(ent_table, rel_table, h3, t3, r3, batch):
    b_per_w = batch // NW
    n_chunks = b_per_w // CHUNK
    mesh = plsc.VectorSubcoreMesh(core_axis_name="c", subcore_axis_name="s")

    @functools.partial(
        pl.kernel,
        mesh=mesh,
        out_type=(
            jax.ShapeDtypeStruct((batch, ENT_DIM), jnp.float32),
            jax.ShapeDtypeStruct((batch, ENT_DIM), jnp.float32),
            jax.ShapeDtypeStruct((batch, GATHER_REL), jnp.float32),
        ),
        scratch_types=[
            pltpu.VMEM((n_chunks, CHUNK), jnp.int32),
            pltpu.VMEM((n_chunks, CHUNK), jnp.int32),
            pltpu.VMEM((n_chunks, CHUNK), jnp.int32),
            pltpu.VMEM((CHUNK, ENT_DIM), jnp.float32),
            pltpu.VMEM((CHUNK, ENT_DIM), jnp.float32),
            pltpu.VMEM((CHUNK, GATHER_REL), jnp.float32),
            pltpu.SemaphoreType.DMA,
            pltpu.SemaphoreType.DMA,
            pltpu.SemaphoreType.DMA,
        ],
    )
    def gather_kernel(ent_hbm, rel_hbm, h_hbm, t_hbm, r_hbm,
                      head_out, tail_out, rel_out,
                      hidx, tidx, ridx, hbuf, tbuf, rbuf, sem_h, sem_t, sem_r):
        wid = lax.axis_index("s") * NC + lax.axis_index("c")
        base = wid * b_per_w
        pltpu.sync_copy(h_hbm.at[wid], hidx)
        pltpu.sync_copy(t_hbm.at[wid], tidx)
        pltpu.sync_copy(r_hbm.at[wid], ridx)
        for c in range(n_chunks):
            off = base + c * CHUNK
            cp_h = pltpu.make_async_copy(ent_hbm.at[hidx.at[c]], hbuf, sem_h)
            cp_t = pltpu.make_async_copy(ent_hbm.at[tidx.at[c]], tbuf, sem_t)
            cp_r = pltpu.make_async_copy(rel_hbm.at[ridx.at[c]], rbuf, sem_r)
            cp_h.start()
            cp_t.start()
            cp_r.start()
            cp_h.wait()
            pltpu.sync_copy(hbuf, head_out.at[pl.ds(off, CHUNK)])
            cp_t.wait()
            pltpu.sync_copy(tbuf, tail_out.at[pl.ds(off, CHUNK)])
            cp_r.wait()
            pltpu.sync_copy(rbuf, rel_out.at[pl.ds(off, CHUNK)])

    return gather_kernel(ent_table, rel_table, h3, t3, r3)


def _score_body(head_ref, tail_ref, rel_ref, out_ref):
    head = head_ref[...]
    tail = tail_ref[...]
    rel = rel_ref[...]

    eps = jnp.float32(1e-12)
    hn = head / jnp.maximum(
        jnp.sqrt(jnp.sum(head * head, axis=1, keepdims=True)), eps)
    tn = tail / jnp.maximum(
        jnp.sqrt(jnp.sum(tail * tail, axis=1, keepdims=True)), eps)

    scale = rel[:, 0:ENT_DIM]
    translate = rel[:, ENT_DIM:2 * ENT_DIM]
    theta = rel[:, 2 * ENT_DIM:2 * ENT_DIM + ENT_DIM // 2]
    theta = theta * jnp.float32(PI / EMBEDDING_RANGE)
    cos_t = jnp.cos(theta)
    sin_t = jnp.sin(theta)

    r64 = lax.broadcasted_iota(jnp.int32, (ENT_DIM // 2, ENT_DIM), 0)
    c128 = lax.broadcasted_iota(jnp.int32, (ENT_DIM // 2, ENT_DIM), 1)
    p_dup = (c128 // 2 == r64).astype(jnp.float32)
    ra = lax.broadcasted_iota(jnp.int32, (ENT_DIM, ENT_DIM), 0)
    cb = lax.broadcasted_iota(jnp.int32, (ENT_DIM, ENT_DIM), 1)
    p_swap = (cb == (ra ^ 1)).astype(jnp.float32)

    dot = functools.partial(
        jax.lax.dot_general,
        dimension_numbers=(((1,), (0,)), ((), ())),
        precision=jax.lax.Precision.HIGHEST,
    )
    cos_d = dot(cos_t, p_dup)
    sin_d = dot(sin_t, p_dup)
    tn_sw = dot(tn, p_swap)

    lane = lax.broadcasted_iota(jnp.int32, (1, ENT_DIM), 1)
    sgn = jnp.where(lane % 2 == 0, jnp.float32(-1.0), jnp.float32(1.0))
    rot = cos_d * tn + sgn * sin_d * tn_sw
    out = (rot + translate) * scale

    score = jnp.float32(GAMMA) - jnp.sum(jnp.abs(hn - out), axis=1, keepdims=True)
    out_ref[...] = score


def _tc_score(head, tail, rel, batch):
    bt = 1024
    grid = batch // bt
    return pl.pallas_call(
        _score_body,
        grid=(grid,),
        in_specs=[
            pl.BlockSpec((bt, ENT_DIM), lambda i: (i, 0)),
            pl.BlockSpec((bt, ENT_DIM), lambda i: (i, 0)),
            pl.BlockSpec((bt, GATHER_REL), lambda i: (i, 0)),
        ],
        out_specs=pl.BlockSpec((bt, 1), lambda i: (i, 0)),
        out_shape=jax.ShapeDtypeStruct((batch, 1), jnp.float32),
    )(head, tail, rel)


def kernel(h, r, t, batch_type, ent_table, rel_table):
    batch = h.shape[0]
    b_per_w = batch // NW
    n_chunks = b_per_w // CHUNK
    h3 = h.reshape(NW, n_chunks, CHUNK)
    t3 = t.reshape(NW, n_chunks, CHUNK)
    r3 = r.reshape(NW, n_chunks, CHUNK)
    head, tail, rel = _sc_gather(ent_table, rel_table, h3, t3, r3, batch)
    return _tc_score(head, tail, rel, batch)


# speedup vs baseline: 2.3818x; 1.0718x over previous
import functools

import jax
import jax.numpy as jnp
from jax import lax
from jax.experimental import pallas as pl
from jax.experimental.pallas import tpu as pltpu
from jax.experimental.pallas import tpu_sc as plsc

ENT_DIM = 128
REL_DIM = 384
EMBEDDING_RANGE = 0.109375
GAMMA = 12.0
PI = 3.141592653589793

NC = 2
NS = 16
NW = NC * NS
CHUNK = 64
NBUF = 3


def _rel_prep_body(rel_ref, out_ref):
    rel = rel_ref[...]
    theta = rel[:, 2 * ENT_DIM:2 * ENT_DIM + ENT_DIM // 2]
    theta = theta * jnp.float32(PI / EMBEDDING_RANGE)
    out_ref[...] = jnp.concatenate(
        [rel[:, :2 * ENT_DIM], jnp.cos(theta), jnp.sin(theta)], axis=1)


def _rel_prep(rel_table):
    n = rel_table.shape[0]
    return pl.pallas_call(
        _rel_prep_body,
        out_shape=jax.ShapeDtypeStruct((n, REL_DIM), jnp.float32),
    )(rel_table)


def _sc_gather(ent_table, rel_proc, h3, t3, r3, batch):
    b_per_w = batch // NW
    n_chunks = b_per_w // CHUNK
    mesh = plsc.VectorSubcoreMesh(core_axis_name="c", subcore_axis_name="s")

    @functools.partial(
        pl.kernel,
        mesh=mesh,
        out_type=(
            jax.ShapeDtypeStruct((batch, ENT_DIM), jnp.float32),
            jax.ShapeDtypeStruct((batch, ENT_DIM), jnp.float32),
            jax.ShapeDtypeStruct((batch, REL_DIM), jnp.float32),
        ),
        scratch_types=[
            pltpu.VMEM((n_chunks, CHUNK), jnp.int32),
            pltpu.VMEM((n_chunks, CHUNK), jnp.int32),
            pltpu.VMEM((n_chunks, CHUNK), jnp.int32),
            pltpu.VMEM((NBUF, CHUNK, ENT_DIM), jnp.float32),
            pltpu.VMEM((NBUF, CHUNK, ENT_DIM), jnp.float32),
            pltpu.VMEM((NBUF, CHUNK, REL_DIM), jnp.float32),
        ]
        + [pltpu.SemaphoreType.DMA] * (6 * NBUF),
    )
    def gather_kernel(ent_hbm, rel_hbm, h_hbm, t_hbm, r_hbm,
                      head_out, tail_out, rel_out,
                      hidx, tidx, ridx, hbuf, tbuf, rbuf, *sems):
        g_sems = sems[:3 * NBUF]
        w_sems = sems[3 * NBUF:]
        wid = lax.axis_index("s") * NC + lax.axis_index("c")
        base = wid * b_per_w
        pltpu.sync_copy(h_hbm.at[wid], hidx)
        pltpu.sync_copy(t_hbm.at[wid], tidx)
        pltpu.sync_copy(r_hbm.at[wid], ridx)

        def start_gather(c, b):
            pltpu.make_async_copy(
                ent_hbm.at[hidx.at[c]], hbuf.at[b], g_sems[b]).start()
            pltpu.make_async_copy(
                ent_hbm.at[tidx.at[c]], tbuf.at[b], g_sems[NBUF + b]).start()
            pltpu.make_async_copy(
                rel_hbm.at[ridx.at[c]], rbuf.at[b], g_sems[2 * NBUF + b]).start()

        def wait_gather(b):
            pltpu.make_async_copy(
                ent_hbm.at[hidx.at[0]], hbuf.at[b], g_sems[b]).wait()
            pltpu.make_async_copy(
                ent_hbm.at[tidx.at[0]], tbuf.at[b], g_sems[NBUF + b]).wait()
            pltpu.make_async_copy(
                rel_hbm.at[ridx.at[0]], rbuf.at[b], g_sems[2 * NBUF + b]).wait()

        def make_wb(c, b):
            off = base + c * CHUNK
            return (
                pltpu.make_async_copy(
                    hbuf.at[b], head_out.at[pl.ds(off, CHUNK)], w_sems[b]),
                pltpu.make_async_copy(
                    tbuf.at[b], tail_out.at[pl.ds(off, CHUNK)], w_sems[NBUF + b]),
                pltpu.make_async_copy(
                    rbuf.at[b], rel_out.at[pl.ds(off, CHUNK)], w_sems[2 * NBUF + b]),
            )

        for c in range(min(NBUF, n_chunks)):
            start_gather(c, c)
        for c in range(n_chunks):
            b = c % NBUF
            wait_gather(b)
            wbs = make_wb(c, b)
            for wb in wbs:
                wb.start()
            nxt = c + NBUF
            if nxt < n_chunks:
                for wb in wbs:
                    wb.wait()
                start_gather(nxt, b)
            else:
                for wb in wbs:
                    wb.wait()

    return gather_kernel(ent_table, rel_proc, h3, t3, r3)


def _score_body(head_ref, tail_ref, rel_ref, out_ref):
    head = head_ref[...]
    tail = tail_ref[...]
    rel = rel_ref[...]

    eps = jnp.float32(1e-12)
    hn = head / jnp.maximum(
        jnp.sqrt(jnp.sum(head * head, axis=1, keepdims=True)), eps)
    tn = tail / jnp.maximum(
        jnp.sqrt(jnp.sum(tail * tail, axis=1, keepdims=True)), eps)

    scale = rel[:, 0:ENT_DIM]
    translate = rel[:, ENT_DIM:2 * ENT_DIM]
    cos_t = rel[:, 2 * ENT_DIM:2 * ENT_DIM + ENT_DIM // 2]
    sin_t = rel[:, 2 * ENT_DIM + ENT_DIM // 2:REL_DIM]

    r64 = lax.broadcasted_iota(jnp.int32, (ENT_DIM // 2, ENT_DIM), 0)
    c128 = lax.broadcasted_iota(jnp.int32, (ENT_DIM // 2, ENT_DIM), 1)
    p_dup = (c128 // 2 == r64).astype(jnp.float32)
    ra = lax.broadcasted_iota(jnp.int32, (ENT_DIM, ENT_DIM), 0)
    cb = lax.broadcasted_iota(jnp.int32, (ENT_DIM, ENT_DIM), 1)
    p_swap = (cb == (ra ^ 1)).astype(jnp.float32)

    dot = functools.partial(
        jax.lax.dot_general,
        dimension_numbers=(((1,), (0,)), ((), ())),
        precision=jax.lax.Precision.HIGHEST,
    )
    cos_d = dot(cos_t, p_dup)
    sin_d = dot(sin_t, p_dup)
    tn_sw = dot(tn, p_swap)

    lane = lax.broadcasted_iota(jnp.int32, (1, ENT_DIM), 1)
    sgn = jnp.where(lane % 2 == 0, jnp.float32(-1.0), jnp.float32(1.0))
    rot = cos_d * tn + sgn * sin_d * tn_sw
    out = (rot + translate) * scale

    score = jnp.float32(GAMMA) - jnp.sum(jnp.abs(hn - out), axis=1, keepdims=True)
    out_ref[...] = score


def _tc_score(head, tail, rel, batch):
    bt = 1024
    grid = batch // bt
    return pl.pallas_call(
        _score_body,
        grid=(grid,),
        in_specs=[
            pl.BlockSpec((bt, ENT_DIM), lambda i: (i, 0)),
            pl.BlockSpec((bt, ENT_DIM), lambda i: (i, 0)),
            pl.BlockSpec((bt, REL_DIM), lambda i: (i, 0)),
        ],
        out_specs=pl.BlockSpec((bt, 1), lambda i: (i, 0)),
        out_shape=jax.ShapeDtypeStruct((batch, 1), jnp.float32),
    )(head, tail, rel)


def kernel(h, r, t, batch_type, ent_table, rel_table):
    batch = h.shape[0]
    b_per_w = batch // NW
    n_chunks = b_per_w // CHUNK
    rel_proc = _rel_prep(rel_table)
    h3 = h.reshape(NW, n_chunks, CHUNK)
    t3 = t.reshape(NW, n_chunks, CHUNK)
    r3 = r.reshape(NW, n_chunks, CHUNK)
    head, tail, rel = _sc_gather(ent_table, rel_proc, h3, t3, r3, batch)
    return _tc_score(head, tail, rel, batch)


# speedup vs baseline: 3.6104x; 1.5158x over previous
import functools

import jax
import jax.numpy as jnp
from jax import lax
from jax.experimental import pallas as pl
from jax.experimental.pallas import tpu as pltpu
from jax.experimental.pallas import tpu_sc as plsc

ENT_DIM = 128
REL_DIM = 384
EMBEDDING_RANGE = 0.109375
GAMMA = 12.0
PI = 3.141592653589793

NC = 2
NS = 16
NW = NC * NS
CHUNK = 64
NBUF = 2
L = 16


def _rel_prep_body(rel_ref, out_ref):
    rel = rel_ref[...]
    theta = rel[:, 2 * ENT_DIM:2 * ENT_DIM + ENT_DIM // 2]
    theta = theta * jnp.float32(PI / EMBEDDING_RANGE)
    out_ref[...] = jnp.concatenate(
        [rel[:, :2 * ENT_DIM], jnp.cos(theta), jnp.sin(theta)], axis=1)


def _rel_prep(rel_table):
    n = rel_table.shape[0]
    return pl.pallas_call(
        _rel_prep_body,
        out_shape=jax.ShapeDtypeStruct((n, REL_DIM), jnp.float32),
    )(rel_table)


def _vec_rsqrt(s):
    one = jnp.full((L,), 1.0, jnp.float32)
    y = one / jnp.maximum(s, one)
    half = jnp.full((L,), 0.5, jnp.float32) * s
    c15 = jnp.full((L,), 1.5, jnp.float32)
    for _ in range(10):
        y = y * (c15 - half * y * y)
    return y


def _lane_gather(vec, idx):
    return lax.gather(
        vec, idx[:, None],
        dimension_numbers=lax.GatherDimensionNumbers(
            offset_dims=(), collapsed_slice_dims=(0,), start_index_map=(0,)),
        slice_sizes=(1,),
        mode=lax.GatherScatterMode.PROMISE_IN_BOUNDS)


def _lane_sum(v, lane_iota):
    for sh in (8, 4, 2, 1):
        idx = lane_iota ^ jnp.full((L,), sh, jnp.int32)
        v = v + _lane_gather(v, idx)
    return v


def _sc_score(ent_table, rel_proc, h3, t3, r3, batch):
    b_per_w = batch // NW
    n_chunks = b_per_w // CHUNK
    mesh = plsc.VectorSubcoreMesh(core_axis_name="c", subcore_axis_name="s")

    @functools.partial(
        pl.kernel,
        mesh=mesh,
        out_type=jax.ShapeDtypeStruct((NW, b_per_w), jnp.float32),
        scratch_types=[
            pltpu.VMEM((n_chunks, CHUNK), jnp.int32),
            pltpu.VMEM((n_chunks, CHUNK), jnp.int32),
            pltpu.VMEM((n_chunks, CHUNK), jnp.int32),
            pltpu.VMEM((NBUF, CHUNK, ENT_DIM), jnp.float32),
            pltpu.VMEM((NBUF, CHUNK, ENT_DIM), jnp.float32),
            pltpu.VMEM((NBUF, CHUNK, REL_DIM), jnp.float32),
            pltpu.VMEM((b_per_w,), jnp.float32),
        ]
        + [pltpu.SemaphoreType.DMA] * (3 * NBUF),
    )
    def score_kernel(ent_hbm, rel_hbm, h_hbm, t_hbm, r_hbm, out_hbm,
                     hidx, tidx, ridx, hbuf, tbuf, rbuf, sbuf, *sems):
        wid = lax.axis_index("s") * NC + lax.axis_index("c")
        pltpu.sync_copy(h_hbm.at[wid], hidx)
        pltpu.sync_copy(t_hbm.at[wid], tidx)
        pltpu.sync_copy(r_hbm.at[wid], ridx)

        def start_gather(c, b):
            pltpu.make_async_copy(
                ent_hbm.at[hidx.at[c]], hbuf.at[b], sems[b]).start()
            pltpu.make_async_copy(
                ent_hbm.at[tidx.at[c]], tbuf.at[b], sems[NBUF + b]).start()
            pltpu.make_async_copy(
                rel_hbm.at[ridx.at[c]], rbuf.at[b], sems[2 * NBUF + b]).start()

        def wait_gather(b):
            pltpu.make_async_copy(
                ent_hbm.at[hidx.at[0]], hbuf.at[b], sems[b]).wait()
            pltpu.make_async_copy(
                ent_hbm.at[tidx.at[0]], tbuf.at[b], sems[NBUF + b]).wait()
            pltpu.make_async_copy(
                rel_hbm.at[ridx.at[0]], rbuf.at[b], sems[2 * NBUF + b]).wait()

        lane_iota = lax.iota(jnp.int32, L)
        one_i = jnp.full((L,), 1, jnp.int32)
        swap_idx = lane_iota ^ one_i
        dup_lo = lax.convert_element_type(
            lax.convert_element_type(lane_iota, jnp.float32)
            * jnp.full((L,), 0.5, jnp.float32), jnp.int32)
        dup_hi = dup_lo + jnp.full((L,), 8, jnp.int32)
        sgn = jnp.where((lane_iota & one_i) == jnp.zeros((L,), jnp.int32),
                        jnp.full((L,), -1.0, jnp.float32),
                        jnp.full((L,), 1.0, jnp.float32))
        eps = jnp.full((L,), 1e-12, jnp.float32)
        gamma = jnp.full((L,), GAMMA, jnp.float32)

        def row_score(hrow, trow, rrow):
            hv, tv = [], []
            hh = jnp.zeros((L,), jnp.float32)
            tt = jnp.zeros((L,), jnp.float32)
            for k in range(ENT_DIM // L):
                hk = hrow[pl.ds(k * L, L)]
                tk = trow[pl.ds(k * L, L)]
                hv.append(hk)
                tv.append(tk)
                hh = hh + hk * hk
                tt = tt + tk * tk
            sh = _lane_sum(hh, lane_iota)
            st = _lane_sum(tt, lane_iota)
            one = jnp.full((L,), 1.0, jnp.float32)
            inv_h = one / jnp.maximum(sh * _vec_rsqrt(sh), eps)
            inv_t = one / jnp.maximum(st * _vec_rsqrt(st), eps)
            cosv = [rrow[pl.ds(2 * ENT_DIM + j * L, L)] for j in range(4)]
            sinv = [rrow[pl.ds(2 * ENT_DIM + 64 + j * L, L)] for j in range(4)]
            acc = jnp.zeros((L,), jnp.float32)
            for k in range(ENT_DIM // L):
                tn = tv[k] * inv_t
                tsw = _lane_gather(tn, swap_idx)
                dup = dup_lo if k % 2 == 0 else dup_hi
                cos_k = _lane_gather(cosv[k // 2], dup)
                sin_k = _lane_gather(sinv[k // 2], dup) * sgn
                rot = cos_k * tn + sin_k * tsw
                out = (rot + rrow[pl.ds(ENT_DIM + k * L, L)]) \
                    * rrow[pl.ds(k * L, L)]
                acc = acc + jnp.abs(hv[k] * inv_h - out)
            return gamma - _lane_sum(acc, lane_iota)

        for c in range(min(NBUF, n_chunks)):
            start_gather(c, c)
        for c in range(n_chunks):
            b = c % NBUF
            wait_gather(b)
            hb, tb, rb = hbuf.at[b], tbuf.at[b], rbuf.at[b]

            @plsc.parallel_loop(0, CHUNK // L, unroll=2)
            def _groups(g, hb=hb, tb=tb, rb=rb, c=c):
                def one_row(j, svec):
                    i = g * L + j
                    score = row_score(hb.at[i], tb.at[i], rb.at[i])
                    mask = lane_iota == lax.broadcast(j, (L,))
                    return jnp.where(mask, score, svec)
                svec = lax.fori_loop(
                    0, L, one_row, jnp.zeros((L,), jnp.float32))
                sbuf[pl.ds(c * CHUNK + g * L, L)] = svec

            nxt = c + NBUF
            if nxt < n_chunks:
                start_gather(nxt, b)
        pltpu.sync_copy(sbuf, out_hbm.at[wid])

    return score_kernel(ent_table, rel_proc, h3, t3, r3)


def kernel(h, r, t, batch_type, ent_table, rel_table):
    batch = h.shape[0]
    b_per_w = batch // NW
    n_chunks = b_per_w // CHUNK
    rel_proc = _rel_prep(rel_table)
    h3 = h.reshape(NW, n_chunks, CHUNK)
    t3 = t.reshape(NW, n_chunks, CHUNK)
    r3 = r.reshape(NW, n_chunks, CHUNK)
    scores = _sc_score(ent_table, rel_proc, h3, t3, r3, batch)
    return scores.reshape(batch, 1)


# speedup vs baseline: 3.8206x; 1.0582x over previous
import functools

import jax
import jax.numpy as jnp
from jax import lax
from jax.experimental import pallas as pl
from jax.experimental.pallas import tpu as pltpu
from jax.experimental.pallas import tpu_sc as plsc

ENT_DIM = 128
REL_DIM = 384
EMBEDDING_RANGE = 0.109375
GAMMA = 12.0
PI = 3.141592653589793

NC = 2
NS = 16
NW = NC * NS
CHUNK = 64
NBUF = 2
L = 16


def _rel_prep_body(rel_ref, out_ref):
    rel = rel_ref[...]
    scale = rel[:, 0:ENT_DIM]
    translate = rel[:, ENT_DIM:2 * ENT_DIM]
    theta = rel[:, 2 * ENT_DIM:2 * ENT_DIM + ENT_DIM // 2]
    theta = theta * jnp.float32(PI / EMBEDDING_RANGE)
    r64 = lax.broadcasted_iota(jnp.int32, (ENT_DIM // 2, ENT_DIM), 0)
    c128 = lax.broadcasted_iota(jnp.int32, (ENT_DIM // 2, ENT_DIM), 1)
    p_dup = (c128 // 2 == r64).astype(jnp.float32)
    dot = functools.partial(
        jax.lax.dot_general,
        dimension_numbers=(((1,), (0,)), ((), ())),
        precision=jax.lax.Precision.HIGHEST,
    )
    cos_d = dot(jnp.cos(theta), p_dup)
    sin_d = dot(jnp.sin(theta), p_dup)
    lane = lax.broadcasted_iota(jnp.int32, (1, ENT_DIM), 1)
    sgn = jnp.where(lane % 2 == 0, jnp.float32(-1.0), jnp.float32(1.0))
    out_ref[...] = jnp.concatenate(
        [cos_d * scale, sgn * sin_d * scale, translate * scale], axis=1)


def _rel_prep(rel_table):
    n = rel_table.shape[0]
    return pl.pallas_call(
        _rel_prep_body,
        out_shape=jax.ShapeDtypeStruct((n, REL_DIM), jnp.float32),
    )(rel_table)


def _vec_rsqrt(s):
    one = jnp.full((L,), 1.0, jnp.float32)
    y = (one + one) / (one + s)
    half = jnp.full((L,), 0.5, jnp.float32) * s
    c15 = jnp.full((L,), 1.5, jnp.float32)
    for _ in range(7):
        y = y * (c15 - half * y * y)
    return y


def _lane_gather(vec, idx):
    return lax.gather(
        vec, idx[:, None],
        dimension_numbers=lax.GatherDimensionNumbers(
            offset_dims=(), collapsed_slice_dims=(0,), start_index_map=(0,)),
        slice_sizes=(1,),
        mode=lax.GatherScatterMode.PROMISE_IN_BOUNDS)


def _lane_sum(v, lane_iota):
    for sh in (8, 4, 2, 1):
        idx = lane_iota ^ jnp.full((L,), sh, jnp.int32)
        v = v + _lane_gather(v, idx)
    return v


def _sc_score(ent_table, rel_proc, h3, t3, r3, batch):
    b_per_w = batch // NW
    n_chunks = b_per_w // CHUNK
    mesh = plsc.VectorSubcoreMesh(core_axis_name="c", subcore_axis_name="s")

    @functools.partial(
        pl.kernel,
        mesh=mesh,
        out_type=jax.ShapeDtypeStruct((NW, b_per_w), jnp.float32),
        scratch_types=[
            pltpu.VMEM((n_chunks, CHUNK), jnp.int32),
            pltpu.VMEM((n_chunks, CHUNK), jnp.int32),
            pltpu.VMEM((n_chunks, CHUNK), jnp.int32),
            pltpu.VMEM((NBUF, CHUNK, ENT_DIM), jnp.float32),
            pltpu.VMEM((NBUF, CHUNK, ENT_DIM), jnp.float32),
            pltpu.VMEM((NBUF, CHUNK, REL_DIM), jnp.float32),
            pltpu.VMEM((b_per_w,), jnp.float32),
        ]
        + [pltpu.SemaphoreType.DMA] * (3 * NBUF),
    )
    def score_kernel(ent_hbm, rel_hbm, h_hbm, t_hbm, r_hbm, out_hbm,
                     hidx, tidx, ridx, hbuf, tbuf, rbuf, sbuf, *sems):
        wid = lax.axis_index("s") * NC + lax.axis_index("c")
        pltpu.sync_copy(h_hbm.at[wid], hidx)
        pltpu.sync_copy(t_hbm.at[wid], tidx)
        pltpu.sync_copy(r_hbm.at[wid], ridx)

        def start_gather(c, b):
            pltpu.make_async_copy(
                ent_hbm.at[hidx.at[c]], hbuf.at[b], sems[b]).start()
            pltpu.make_async_copy(
                ent_hbm.at[tidx.at[c]], tbuf.at[b], sems[NBUF + b]).start()
            pltpu.make_async_copy(
                rel_hbm.at[ridx.at[c]], rbuf.at[b], sems[2 * NBUF + b]).start()

        def wait_gather(b):
            pltpu.make_async_copy(
                ent_hbm.at[hidx.at[0]], hbuf.at[b], sems[b]).wait()
            pltpu.make_async_copy(
                ent_hbm.at[tidx.at[0]], tbuf.at[b], sems[NBUF + b]).wait()
            pltpu.make_async_copy(
                rel_hbm.at[ridx.at[0]], rbuf.at[b], sems[2 * NBUF + b]).wait()

        lane_iota = lax.iota(jnp.int32, L)
        one_i = jnp.full((L,), 1, jnp.int32)
        swap_idx = lane_iota ^ one_i
        eps = jnp.full((L,), 1e-12, jnp.float32)
        gamma = jnp.full((L,), GAMMA, jnp.float32)

        def row_score(hrow, trow, rrow):
            hv, tv = [], []
            hh = jnp.zeros((L,), jnp.float32)
            tt = jnp.zeros((L,), jnp.float32)
            for k in range(ENT_DIM // L):
                hk = hrow[pl.ds(k * L, L)]
                tk = trow[pl.ds(k * L, L)]
                hv.append(hk)
                tv.append(tk)
                hh = hh + hk * hk
                tt = tt + tk * tk
            sh = _lane_sum(hh, lane_iota)
            st = _lane_sum(tt, lane_iota)
            one = jnp.full((L,), 1.0, jnp.float32)
            inv_h = one / jnp.maximum(sh * _vec_rsqrt(sh), eps)
            inv_t = one / jnp.maximum(st * _vec_rsqrt(st), eps)
            acc = jnp.zeros((L,), jnp.float32)
            for k in range(ENT_DIM // L):
                tn = tv[k] * inv_t
                tsw = _lane_gather(tn, swap_idx)
                out = rrow[pl.ds(k * L, L)] * tn \
                    + rrow[pl.ds(ENT_DIM + k * L, L)] * tsw \
                    + rrow[pl.ds(2 * ENT_DIM + k * L, L)]
                acc = acc + jnp.abs(hv[k] * inv_h - out)
            return gamma - _lane_sum(acc, lane_iota)

        for c in range(min(NBUF, n_chunks)):
            start_gather(c, c)
        for c in range(n_chunks):
            b = c % NBUF
            wait_gather(b)
            hb, tb, rb = hbuf.at[b], tbuf.at[b], rbuf.at[b]

            @plsc.parallel_loop(0, CHUNK // L, unroll=2)
            def _groups(g, hb=hb, tb=tb, rb=rb, c=c):
                def one_row(j, svec):
                    i = g * L + j
                    score = row_score(hb.at[i], tb.at[i], rb.at[i])
                    mask = lane_iota == lax.broadcast(j, (L,))
                    return jnp.where(mask, score, svec)
                svec = lax.fori_loop(
                    0, L, one_row, jnp.zeros((L,), jnp.float32))
                sbuf[pl.ds(c * CHUNK + g * L, L)] = svec

            nxt = c + NBUF
            if nxt < n_chunks:
                start_gather(nxt, b)
        pltpu.sync_copy(sbuf, out_hbm.at[wid])

    return score_kernel(ent_table, rel_proc, h3, t3, r3)


def kernel(h, r, t, batch_type, ent_table, rel_table):
    batch = h.shape[0]
    b_per_w = batch // NW
    n_chunks = b_per_w // CHUNK
    rel_proc = _rel_prep(rel_table)
    h3 = h.reshape(NW, n_chunks, CHUNK)
    t3 = t.reshape(NW, n_chunks, CHUNK)
    r3 = r.reshape(NW, n_chunks, CHUNK)
    scores = _sc_score(ent_table, rel_proc, h3, t3, r3, batch)
    return scores.reshape(batch, 1)
